# Initial kernel scaffold; baseline (speedup 1.0000x reference)
import functools

import jax
import jax.numpy as jnp
from jax import lax
from jax.experimental import pallas as pl
from jax.experimental.pallas import tpu as pltpu
from jax.experimental.pallas import tpu_sc as plsc

F32 = jnp.float32
NG = 10000
NM = 10000
NE = 320000
H = 128

NC = 2
NS = 16
NW = NC * NS

CH = 80
EPW = NE // NW
NCHUNK = EPW // CH
K = 5
ROWS_PW = NG // NS
ZCH = 125


def _mesh():
  return plsc.VectorSubcoreMesh(core_axis_name="c", subcore_axis_name="s",
                                num_cores=NC, num_subcores=NS)


def _proj_body(mesh_ref, grid_ref, w1b_ref, w1c_ref, be1_ref,
               psrc_ref, pdst_ref):
  psrc_ref[...] = jnp.dot(mesh_ref[...], w1b_ref[...],
                          preferred_element_type=F32)
  pdst_ref[...] = jnp.dot(grid_ref[...], w1c_ref[...],
                          preferred_element_type=F32) + be1_ref[...]


def _project(mesh_nodes, grid_nodes, w1b, w1c, be1):
  blk = 2000
  return pl.pallas_call(
      _proj_body,
      grid=(NM // blk,),
      in_specs=[
          pl.BlockSpec((blk, H), lambda i: (i, 0)),
          pl.BlockSpec((blk, H), lambda i: (i, 0)),
          pl.BlockSpec((H, H), lambda i: (0, 0)),
          pl.BlockSpec((H, H), lambda i: (0, 0)),
          pl.BlockSpec((1, H), lambda i: (0, 0)),
      ],
      out_specs=[
          pl.BlockSpec((blk, H), lambda i: (i, 0)),
          pl.BlockSpec((blk, H), lambda i: (i, 0)),
      ],
      out_shape=[jax.ShapeDtypeStruct((NM, H), F32),
                 jax.ShapeDtypeStruct((NG, H), F32)],
  )(mesh_nodes, grid_nodes, w1b, w1c, be1)


def _sc_gather_body(psrc_hbm, pdst_hbm, srcr_hbm, dstr_hbm,
                    gs_hbm, gd_hbm,
                    idxs_v, idxd_v, rows_v, sem_g, sem_w):
  c = lax.axis_index("c")
  s = lax.axis_index("s")
  wid = c * NS + s
  ebase = wid * EPW
  pltpu.sync_copy(srcr_hbm.at[wid], idxs_v)
  pltpu.sync_copy(dstr_hbm.at[wid], idxd_v)

  def one_table(idx_v, table_hbm, out_hbm):
    def superchunk(m, carry):
      gets = []
      for k in range(K):
        j = m * K + k
        cp = pltpu.make_async_copy(table_hbm.at[idx_v.at[j]],
                                   rows_v.at[k], sem_g)
        cp.start()
        gets.append(cp)
      for cp in gets:
        cp.wait()
      puts = []
      for k in range(K):
        j = m * K + k
        cp = pltpu.make_async_copy(rows_v.at[k],
                                   out_hbm.at[pl.ds(ebase + j * CH, CH)],
                                   sem_w)
        cp.start()
        puts.append(cp)
      for cp in puts:
        cp.wait()
      return carry
    lax.fori_loop(0, NCHUNK // K, superchunk, 0)

  one_table(idxs_v, psrc_hbm, gs_hbm)
  one_table(idxd_v, pdst_hbm, gd_hbm)


def _sc_gather(psrc, pdst, src_r, dst_r):
  fn = pl.kernel(
      _sc_gather_body,
      out_type=[jax.ShapeDtypeStruct((NE, H), F32),
                jax.ShapeDtypeStruct((NE, H), F32)],
      mesh=_mesh(),
      scratch_types=[
          pltpu.VMEM((NCHUNK, CH), jnp.int32),
          pltpu.VMEM((NCHUNK, CH), jnp.int32),
          pltpu.VMEM((K, CH, H), F32),
          pltpu.SemaphoreType.DMA,
          pltpu.SemaphoreType.DMA,
      ],
  )
  return fn(psrc, pdst, src_r, dst_r)


def _edge_body(e_ref, gs_ref, gd_ref, w1a_ref, w2_ref, be2_ref,
               gam_ref, bet_ref, out_ref):
  e = e_ref[...]
  h = jnp.dot(e, w1a_ref[...], preferred_element_type=F32)
  h = h + gs_ref[...] + gd_ref[...]
  h = h * jax.nn.sigmoid(h)
  h = jnp.dot(h, w2_ref[...], preferred_element_type=F32) + be2_ref[...]
  mu = jnp.mean(h, axis=-1, keepdims=True)
  var = jnp.mean((h - mu) * (h - mu), axis=-1, keepdims=True)
  out_ref[...] = ((h - mu) * lax.rsqrt(var + 1e-5) * gam_ref[...]
                  + bet_ref[...] + e)


def _edge_mlp(e_feats, gs, gd, w1a, w2, be2, gam, bet):
  blk = 512
  return pl.pallas_call(
      _edge_body,
      grid=(NE // blk,),
      in_specs=[
          pl.BlockSpec((blk, H), lambda i: (i, 0)),
          pl.BlockSpec((blk, H), lambda i: (i, 0)),
          pl.BlockSpec((blk, H), lambda i: (i, 0)),
          pl.BlockSpec((H, H), lambda i: (0, 0)),
          pl.BlockSpec((H, H), lambda i: (0, 0)),
          pl.BlockSpec((1, H), lambda i: (0, 0)),
          pl.BlockSpec((1, H), lambda i: (0, 0)),
          pl.BlockSpec((1, H), lambda i: (0, 0)),
      ],
      out_specs=pl.BlockSpec((blk, H), lambda i: (i, 0)),
      out_shape=jax.ShapeDtypeStruct((NE, H), F32),
      compiler_params=pltpu.CompilerParams(
          dimension_semantics=("arbitrary",)),
  )(e_feats, gs, gd, w1a, w2, be2, gam, bet)


def _sc_scatter_body(ef_hbm, dstw_hbm, out_hbm,
                     acc_sh, val_v, zbuf_v, idx_v, sem_in):
  c = lax.axis_index("c")
  s = lax.axis_index("s")
  wid = c * NS + s
  ebase = wid * EPW

  zero = jnp.zeros((16,), F32)

  def zrow(r, carry):
    for k in range(H // 16):
      zbuf_v[r, pl.ds(16 * k, 16)] = zero
    return carry
  lax.fori_loop(0, ZCH, zrow, 0)
  rbase = s * ROWS_PW
  for z in range(ROWS_PW // ZCH):
    pltpu.sync_copy(zbuf_v, acc_sh.at[pl.ds(rbase + z * ZCH, ZCH)])
  plsc.subcore_barrier()

  pltpu.sync_copy(dstw_hbm.at[wid], idx_v)

  def superchunk(m, carry):
    gets = []
    for k in range(K):
      j = m * K + k
      cp = pltpu.make_async_copy(ef_hbm.at[pl.ds(ebase + j * CH, CH)],
                                 val_v.at[k], sem_in)
      cp.start()
      gets.append(cp)
    for cp in gets:
      cp.wait()
    for k in range(K):
      j = m * K + k
      pltpu.sync_copy(val_v.at[k], acc_sh.at[idx_v.at[j]], add=True)
    return carry
  lax.fori_loop(0, NCHUNK // K, superchunk, 0)
  plsc.subcore_barrier()

  for z in range(ROWS_PW // ZCH):
    pltpu.sync_copy(acc_sh.at[pl.ds(rbase + z * ZCH, ZCH)], zbuf_v)
    pltpu.sync_copy(zbuf_v, out_hbm.at[c, pl.ds(rbase + z * ZCH, ZCH)])


def _sc_scatter(ef, dst_w):
  fn = pl.kernel(
      _sc_scatter_body,
      out_type=jax.ShapeDtypeStruct((NC, NG, H), F32),
      mesh=_mesh(),
      scratch_types=[
          pltpu.VMEM_SHARED((NG, H), F32),
          pltpu.VMEM((K, CH, H), F32),
          pltpu.VMEM((ZCH, H), F32),
          pltpu.VMEM((NCHUNK, CH), jnp.int32),
          pltpu.SemaphoreType.DMA,
      ],
  )
  return fn(ef, dst_w)


def _node_body(g_ref, p0_ref, p1_ref, wn1a_ref, wn1b_ref, bn1_ref,
               wn2_ref, bn2_ref, gam_ref, bet_ref, out_ref):
  g = g_ref[...]
  agg = p0_ref[...] + p1_ref[...]
  h = (jnp.dot(g, wn1a_ref[...], preferred_element_type=F32)
       + jnp.dot(agg, wn1b_ref[...], preferred_element_type=F32)
       + bn1_ref[...])
  h = h * jax.nn.sigmoid(h)
  h = jnp.dot(h, wn2_ref[...], preferred_element_type=F32) + bn2_ref[...]
  mu = jnp.mean(h, axis=-1, keepdims=True)
  var = jnp.mean((h - mu) * (h - mu), axis=-1, keepdims=True)
  out_ref[...] = ((h - mu) * lax.rsqrt(var + 1e-5) * gam_ref[...]
                  + bet_ref[...] + g)


def _node_mlp(grid_nodes, p0, p1, wn1a, wn1b, bn1, wn2, bn2, gam, bet):
  blk = 2000
  return pl.pallas_call(
      _node_body,
      grid=(NG // blk,),
      in_specs=[
          pl.BlockSpec((blk, H), lambda i: (i, 0)),
          pl.BlockSpec((blk, H), lambda i: (i, 0)),
          pl.BlockSpec((blk, H), lambda i: (i, 0)),
          pl.BlockSpec((H, H), lambda i: (0, 0)),
          pl.BlockSpec((H, H), lambda i: (0, 0)),
          pl.BlockSpec((1, H), lambda i: (0, 0)),
          pl.BlockSpec((H, H), lambda i: (0, 0)),
          pl.BlockSpec((1, H), lambda i: (0, 0)),
          pl.BlockSpec((1, H), lambda i: (0, 0)),
          pl.BlockSpec((1, H), lambda i: (0, 0)),
      ],
      out_specs=pl.BlockSpec((blk, H), lambda i: (i, 0)),
      out_shape=jax.ShapeDtypeStruct((NG, H), F32),
  )(grid_nodes, p0, p1, wn1a, wn1b, bn1, wn2, bn2, gam, bet)


def kernel(mesh2grid_edge_features, grid_node_features, mesh_node_features,
           mesh2grid_edge_indices_src, mesh2grid_edge_indices_dst,
           We1, be1, We2, be2, e_gamma, e_beta,
           Wn1, bn1, Wn2, bn2, n_gamma, n_beta):
  w1a = We1[:H]
  w1b = We1[H:2 * H]
  w1c = We1[2 * H:]
  wn1a = Wn1[:H]
  wn1b = Wn1[H:]
  r1 = lambda v: v.reshape(1, H)

  src_r = mesh2grid_edge_indices_src.reshape(NW, NCHUNK, CH)
  dst_r = mesh2grid_edge_indices_dst.reshape(NW, NCHUNK, CH)

  psrc, pdst = _project(mesh_node_features, grid_node_features,
                        w1b, w1c, r1(be1))
  gs, gd = _sc_gather(psrc, pdst, src_r, dst_r)
  ef = _edge_mlp(mesh2grid_edge_features, gs, gd, w1a, We2, r1(be2),
                 r1(e_gamma), r1(e_beta))
  partials = _sc_scatter(ef, dst_r)
  out = _node_mlp(grid_node_features, partials[0], partials[1],
                  wn1a, wn1b, r1(bn1), Wn2, r1(bn2),
                  r1(n_gamma), r1(n_beta))
  return out


# speedup vs baseline: 2.9912x; 2.9912x over previous
import functools

import jax
import jax.numpy as jnp
from jax import lax
from jax.experimental import pallas as pl
from jax.experimental.pallas import tpu as pltpu
from jax.experimental.pallas import tpu_sc as plsc

F32 = jnp.float32
NG = 10000
NM = 10000
NE = 320000
H = 128

NC = 2
NS = 16
NW = NC * NS

CH = 80
EPW = NE // NW
NCHUNK = EPW // CH
K = 5
NGP = 10240
ACC_PW = NGP // NS
SCH = 40
SNCHUNK = EPW // SCH


def _mesh():
  return plsc.VectorSubcoreMesh(core_axis_name="c", subcore_axis_name="s",
                                num_cores=NC, num_subcores=NS)


def _proj_body(mesh_ref, grid_ref, w1b_ref, w1c_ref, be1_ref,
               psrc_ref, pdst_ref):
  psrc_ref[...] = jnp.dot(mesh_ref[...], w1b_ref[...],
                          preferred_element_type=F32)
  pdst_ref[...] = jnp.dot(grid_ref[...], w1c_ref[...],
                          preferred_element_type=F32) + be1_ref[...]


def _project(mesh_nodes, grid_nodes, w1b, w1c, be1):
  blk = 2000
  return pl.pallas_call(
      _proj_body,
      grid=(NM // blk,),
      in_specs=[
          pl.BlockSpec((blk, H), lambda i: (i, 0)),
          pl.BlockSpec((blk, H), lambda i: (i, 0)),
          pl.BlockSpec((H, H), lambda i: (0, 0)),
          pl.BlockSpec((H, H), lambda i: (0, 0)),
          pl.BlockSpec((1, H), lambda i: (0, 0)),
      ],
      out_specs=[
          pl.BlockSpec((blk, H), lambda i: (i, 0)),
          pl.BlockSpec((blk, H), lambda i: (i, 0)),
      ],
      out_shape=[jax.ShapeDtypeStruct((NM, H), F32),
                 jax.ShapeDtypeStruct((NG, H), F32)],
  )(mesh_nodes, grid_nodes, w1b, w1c, be1)


def _sc_gather_body(psrc_hbm, pdst_hbm, srcr_hbm, dstr_hbm,
                    gs_hbm, gd_hbm,
                    idxs_v, idxd_v, rows_v, sem_g, sem_w):
  c = lax.axis_index("c")
  s = lax.axis_index("s")
  wid = c * NS + s
  ebase = wid * EPW
  pltpu.sync_copy(srcr_hbm.at[wid], idxs_v)
  pltpu.sync_copy(dstr_hbm.at[wid], idxd_v)

  def one_table(idx_v, table_hbm, out_hbm):
    def superchunk(m, carry):
      gets = []
      for k in range(K):
        j = m * K + k
        cp = pltpu.make_async_copy(table_hbm.at[idx_v.at[j]],
                                   rows_v.at[k], sem_g)
        cp.start()
        gets.append(cp)
      for cp in gets:
        cp.wait()
      puts = []
      for k in range(K):
        j = m * K + k
        cp = pltpu.make_async_copy(rows_v.at[k],
                                   out_hbm.at[pl.ds(ebase + j * CH, CH)],
                                   sem_w)
        cp.start()
        puts.append(cp)
      for cp in puts:
        cp.wait()
      return carry
    lax.fori_loop(0, NCHUNK // K, superchunk, 0)

  one_table(idxs_v, psrc_hbm, gs_hbm)
  one_table(idxd_v, pdst_hbm, gd_hbm)


def _sc_gather(psrc, pdst, src_r, dst_r):
  fn = pl.kernel(
      _sc_gather_body,
      out_type=[jax.ShapeDtypeStruct((NE, H), F32),
                jax.ShapeDtypeStruct((NE, H), F32)],
      mesh=_mesh(),
      scratch_types=[
          pltpu.VMEM((NCHUNK, CH), jnp.int32),
          pltpu.VMEM((NCHUNK, CH), jnp.int32),
          pltpu.VMEM((K, CH, H), F32),
          pltpu.SemaphoreType.DMA,
          pltpu.SemaphoreType.DMA,
      ],
  )
  return fn(psrc, pdst, src_r, dst_r)


def _edge_body(e_ref, gs_ref, gd_ref, w1a_ref, w2_ref, be2_ref,
               gam_ref, bet_ref, out_ref):
  e = e_ref[...]
  h = jnp.dot(e, w1a_ref[...], preferred_element_type=F32)
  h = h + gs_ref[...] + gd_ref[...]
  h = h * jax.nn.sigmoid(h)
  h = jnp.dot(h, w2_ref[...], preferred_element_type=F32) + be2_ref[...]
  mu = jnp.mean(h, axis=-1, keepdims=True)
  var = jnp.mean((h - mu) * (h - mu), axis=-1, keepdims=True)
  out_ref[...] = ((h - mu) * lax.rsqrt(var + 1e-5) * gam_ref[...]
                  + bet_ref[...] + e)


def _edge_mlp(e_feats, gs, gd, w1a, w2, be2, gam, bet):
  blk = 512
  return pl.pallas_call(
      _edge_body,
      grid=(NE // blk,),
      in_specs=[
          pl.BlockSpec((blk, H), lambda i: (i, 0)),
          pl.BlockSpec((blk, H), lambda i: (i, 0)),
          pl.BlockSpec((blk, H), lambda i: (i, 0)),
          pl.BlockSpec((H, H), lambda i: (0, 0)),
          pl.BlockSpec((H, H), lambda i: (0, 0)),
          pl.BlockSpec((1, H), lambda i: (0, 0)),
          pl.BlockSpec((1, H), lambda i: (0, 0)),
          pl.BlockSpec((1, H), lambda i: (0, 0)),
      ],
      out_specs=pl.BlockSpec((blk, H), lambda i: (i, 0)),
      out_shape=jax.ShapeDtypeStruct((NE, H), F32),
      compiler_params=pltpu.CompilerParams(
          dimension_semantics=("arbitrary",)),
  )(e_feats, gs, gd, w1a, w2, be2, gam, bet)


def _sc_scatter_body(ef_hbm, dstw_hbm, out_hbm,
                     acc_sh, val_v, idx_v, sem_in):
  c = lax.axis_index("c")
  s = lax.axis_index("s")
  wid = c * NS + s
  ebase = wid * EPW

  zero = jnp.zeros((16,), F32)

  def zrow(r, carry):
    for k in range(H // 16):
      val_v[0, r, pl.ds(16 * k, 16)] = zero
    return carry
  lax.fori_loop(0, SCH, zrow, 0)
  rbase = s * ACC_PW
  for z in range(ACC_PW // SCH):
    pltpu.sync_copy(val_v.at[0], acc_sh.at[pl.ds(rbase + z * SCH, SCH)])
  plsc.subcore_barrier()

  def superchunk(m, carry):
    icp = pltpu.make_async_copy(dstw_hbm.at[wid, m], idx_v, sem_in)
    icp.start()
    gets = []
    for k in range(K):
      j = m * K + k
      cp = pltpu.make_async_copy(ef_hbm.at[pl.ds(ebase + j * SCH, SCH)],
                                 val_v.at[k], sem_in)
      cp.start()
      gets.append(cp)
    icp.wait()
    for cp in gets:
      cp.wait()
    for k in range(K):
      pltpu.sync_copy(val_v.at[k], acc_sh.at[idx_v.at[k]], add=True)
    return carry
  lax.fori_loop(0, SNCHUNK // K, superchunk, 0)
  plsc.subcore_barrier()

  for z in range(ACC_PW // SCH):
    pltpu.sync_copy(acc_sh.at[pl.ds(rbase + z * SCH, SCH)], val_v.at[0])
    pltpu.sync_copy(val_v.at[0], out_hbm.at[c, pl.ds(rbase + z * SCH, SCH)])


def _sc_scatter(ef, dst_w):
  fn = pl.kernel(
      _sc_scatter_body,
      out_type=jax.ShapeDtypeStruct((NC, NGP, H), F32),
      mesh=_mesh(),
      scratch_types=[
          pltpu.VMEM_SHARED((NGP, H), F32),
          pltpu.VMEM((K, SCH, H), F32),
          pltpu.VMEM((K, SCH), jnp.int32),
          pltpu.SemaphoreType.DMA,
      ],
  )
  return fn(ef, dst_w)


def _node_body(g_ref, p0_ref, p1_ref, wn1a_ref, wn1b_ref, bn1_ref,
               wn2_ref, bn2_ref, gam_ref, bet_ref, out_ref):
  g = g_ref[...]
  agg = p0_ref[0] + p1_ref[0]
  h = (jnp.dot(g, wn1a_ref[...], preferred_element_type=F32)
       + jnp.dot(agg, wn1b_ref[...], preferred_element_type=F32)
       + bn1_ref[...])
  h = h * jax.nn.sigmoid(h)
  h = jnp.dot(h, wn2_ref[...], preferred_element_type=F32) + bn2_ref[...]
  mu = jnp.mean(h, axis=-1, keepdims=True)
  var = jnp.mean((h - mu) * (h - mu), axis=-1, keepdims=True)
  out_ref[...] = ((h - mu) * lax.rsqrt(var + 1e-5) * gam_ref[...]
                  + bet_ref[...] + g)


def _node_mlp(grid_nodes, partials, wn1a, wn1b, bn1, wn2, bn2, gam, bet):
  blk = 2000
  return pl.pallas_call(
      _node_body,
      grid=(NG // blk,),
      in_specs=[
          pl.BlockSpec((blk, H), lambda i: (i, 0)),
          pl.BlockSpec((1, blk, H), lambda i: (0, i, 0)),
          pl.BlockSpec((1, blk, H), lambda i: (1, i, 0)),
          pl.BlockSpec((H, H), lambda i: (0, 0)),
          pl.BlockSpec((H, H), lambda i: (0, 0)),
          pl.BlockSpec((1, H), lambda i: (0, 0)),
          pl.BlockSpec((H, H), lambda i: (0, 0)),
          pl.BlockSpec((1, H), lambda i: (0, 0)),
          pl.BlockSpec((1, H), lambda i: (0, 0)),
          pl.BlockSpec((1, H), lambda i: (0, 0)),
      ],
      out_specs=pl.BlockSpec((blk, H), lambda i: (i, 0)),
      out_shape=jax.ShapeDtypeStruct((NG, H), F32),
  )(grid_nodes, partials, partials, wn1a, wn1b, bn1, wn2, bn2, gam, bet)


def kernel(mesh2grid_edge_features, grid_node_features, mesh_node_features,
           mesh2grid_edge_indices_src, mesh2grid_edge_indices_dst,
           We1, be1, We2, be2, e_gamma, e_beta,
           Wn1, bn1, Wn2, bn2, n_gamma, n_beta):
  w1a = We1[:H]
  w1b = We1[H:2 * H]
  w1c = We1[2 * H:]
  wn1a = Wn1[:H]
  wn1b = Wn1[H:]
  r1 = lambda v: v.reshape(1, H)

  src_r = mesh2grid_edge_indices_src.reshape(NW, NCHUNK, CH)
  dst_r = mesh2grid_edge_indices_dst.reshape(NW, NCHUNK, CH)
  dst_w = mesh2grid_edge_indices_dst.reshape(NW, SNCHUNK // K, K, SCH)

  psrc, pdst = _project(mesh_node_features, grid_node_features,
                        w1b, w1c, r1(be1))
  gs, gd = _sc_gather(psrc, pdst, src_r, dst_r)
  ef = _edge_mlp(mesh2grid_edge_features, gs, gd, w1a, We2, r1(be2),
                 r1(e_gamma), r1(e_beta))
  partials = _sc_scatter(ef, dst_w)
  out = _node_mlp(grid_node_features, partials,
                  wn1a, wn1b, r1(bn1), Wn2, r1(bn2),
                  r1(n_gamma), r1(n_beta))
  return out


# speedup vs baseline: 3.8662x; 1.2926x over previous
import functools

import jax
import jax.numpy as jnp
from jax import lax
from jax.experimental import pallas as pl
from jax.experimental.pallas import tpu as pltpu
from jax.experimental.pallas import tpu_sc as plsc

F32 = jnp.float32
NG = 10000
NM = 10000
NE = 320000
H = 128

NC = 2
NS = 16
NW = NC * NS

CH = 80
EPW = NE // NW
NCHUNK = EPW // CH
K = 5
NGP = 10240
ACC_PW = NGP // NS
SCH = 40
SNCHUNK = EPW // SCH


def _mesh():
  return plsc.VectorSubcoreMesh(core_axis_name="c", subcore_axis_name="s",
                                num_cores=NC, num_subcores=NS)


def _proj_body(mesh_ref, grid_ref, w1b_ref, w1c_ref, be1_ref,
               psrc_ref, pdst_ref):
  psrc_ref[...] = jnp.dot(mesh_ref[...], w1b_ref[...],
                          preferred_element_type=F32)
  pdst_ref[...] = jnp.dot(grid_ref[...], w1c_ref[...],
                          preferred_element_type=F32) + be1_ref[...]


def _project(mesh_nodes, grid_nodes, w1b, w1c, be1):
  blk = 2000
  return pl.pallas_call(
      _proj_body,
      grid=(NM // blk,),
      in_specs=[
          pl.BlockSpec((blk, H), lambda i: (i, 0)),
          pl.BlockSpec((blk, H), lambda i: (i, 0)),
          pl.BlockSpec((H, H), lambda i: (0, 0)),
          pl.BlockSpec((H, H), lambda i: (0, 0)),
          pl.BlockSpec((1, H), lambda i: (0, 0)),
      ],
      out_specs=[
          pl.BlockSpec((blk, H), lambda i: (i, 0)),
          pl.BlockSpec((blk, H), lambda i: (i, 0)),
      ],
      out_shape=[jax.ShapeDtypeStruct((NM, H), F32),
                 jax.ShapeDtypeStruct((NG, H), F32)],
  )(mesh_nodes, grid_nodes, w1b, w1c, be1)


def _sc_gather_body(psrc_hbm, pdst_hbm, srcr_hbm, dstr_hbm,
                    gs_hbm, gd_hbm,
                    idxs_v, idxd_v, rows_v, sem_g, sem_w):
  c = lax.axis_index("c")
  s = lax.axis_index("s")
  wid = c * NS + s
  ebase = wid * EPW
  pltpu.sync_copy(srcr_hbm.at[wid], idxs_v)
  pltpu.sync_copy(dstr_hbm.at[wid], idxd_v)

  def one_table(idx_v, table_hbm, out_hbm):
    def superchunk(m, carry):
      gets = []
      for k in range(K):
        j = m * K + k
        cp = pltpu.make_async_copy(table_hbm.at[idx_v.at[j]],
                                   rows_v.at[k], sem_g)
        cp.start()
        gets.append(cp)
      for cp in gets:
        cp.wait()
      puts = []
      for k in range(K):
        j = m * K + k
        cp = pltpu.make_async_copy(rows_v.at[k],
                                   out_hbm.at[pl.ds(ebase + j * CH, CH)],
                                   sem_w)
        cp.start()
        puts.append(cp)
      for cp in puts:
        cp.wait()
      return carry
    lax.fori_loop(0, NCHUNK // K, superchunk, 0)

  one_table(idxs_v, psrc_hbm, gs_hbm)
  one_table(idxd_v, pdst_hbm, gd_hbm)


def _sc_gather(psrc, pdst, src_r, dst_r):
  fn = pl.kernel(
      _sc_gather_body,
      out_type=[jax.ShapeDtypeStruct((NE, H), F32),
                jax.ShapeDtypeStruct((NE, H), F32)],
      mesh=_mesh(),
      scratch_types=[
          pltpu.VMEM((NCHUNK, CH), jnp.int32),
          pltpu.VMEM((NCHUNK, CH), jnp.int32),
          pltpu.VMEM((K, CH, H), F32),
          pltpu.SemaphoreType.DMA,
          pltpu.SemaphoreType.DMA,
      ],
  )
  return fn(psrc, pdst, src_r, dst_r)


def _edge_body(e_ref, gs_ref, gd_ref, w1a_ref, w2_ref, be2_ref,
               gam_ref, bet_ref, out_ref):
  e = e_ref[...]
  h = jnp.dot(e.astype(jnp.bfloat16), w1a_ref[...],
              preferred_element_type=F32)
  h = h + gs_ref[...] + gd_ref[...]
  h = h * jax.nn.sigmoid(h)
  h = jnp.dot(h.astype(jnp.bfloat16), w2_ref[...],
              preferred_element_type=F32) + be2_ref[...]
  mu = jnp.mean(h, axis=-1, keepdims=True)
  var = jnp.mean((h - mu) * (h - mu), axis=-1, keepdims=True)
  out_ref[...] = ((h - mu) * lax.rsqrt(var + 1e-5) * gam_ref[...]
                  + bet_ref[...] + e)


def _edge_mlp(e_feats, gs, gd, w1a, w2, be2, gam, bet):
  blk = 1280
  return pl.pallas_call(
      _edge_body,
      grid=(NE // blk,),
      in_specs=[
          pl.BlockSpec((blk, H), lambda i: (i, 0)),
          pl.BlockSpec((blk, H), lambda i: (i, 0)),
          pl.BlockSpec((blk, H), lambda i: (i, 0)),
          pl.BlockSpec((H, H), lambda i: (0, 0)),
          pl.BlockSpec((H, H), lambda i: (0, 0)),
          pl.BlockSpec((1, H), lambda i: (0, 0)),
          pl.BlockSpec((1, H), lambda i: (0, 0)),
          pl.BlockSpec((1, H), lambda i: (0, 0)),
      ],
      out_specs=pl.BlockSpec((blk, H), lambda i: (i, 0)),
      out_shape=jax.ShapeDtypeStruct((NE, H), F32),
      compiler_params=pltpu.CompilerParams(
          dimension_semantics=("arbitrary",)),
  )(e_feats, gs, gd, w1a, w2, be2, gam, bet)


def _sc_scatter_body(ef_hbm, dstw_hbm, out_hbm,
                     acc_sh, val_v, idx_v, sem_in):
  c = lax.axis_index("c")
  s = lax.axis_index("s")
  wid = c * NS + s
  ebase = wid * EPW

  zero = jnp.zeros((16,), F32)

  def zrow(r, carry):
    for k in range(H // 16):
      val_v[0, r, pl.ds(16 * k, 16)] = zero
    return carry
  lax.fori_loop(0, SCH, zrow, 0)
  rbase = s * ACC_PW
  for z in range(ACC_PW // SCH):
    pltpu.sync_copy(val_v.at[0], acc_sh.at[pl.ds(rbase + z * SCH, SCH)])
  plsc.subcore_barrier()

  def superchunk(m, carry):
    icp = pltpu.make_async_copy(dstw_hbm.at[wid, m], idx_v, sem_in)
    icp.start()
    gets = []
    for k in range(K):
      j = m * K + k
      cp = pltpu.make_async_copy(ef_hbm.at[pl.ds(ebase + j * SCH, SCH)],
                                 val_v.at[k], sem_in)
      cp.start()
      gets.append(cp)
    icp.wait()
    for cp in gets:
      cp.wait()
    for k in range(K):
      pltpu.sync_copy(val_v.at[k], acc_sh.at[idx_v.at[k]], add=True)
    return carry
  lax.fori_loop(0, SNCHUNK // K, superchunk, 0)
  plsc.subcore_barrier()

  for z in range(ACC_PW // SCH):
    pltpu.sync_copy(acc_sh.at[pl.ds(rbase + z * SCH, SCH)], val_v.at[0])
    pltpu.sync_copy(val_v.at[0], out_hbm.at[c, pl.ds(rbase + z * SCH, SCH)])


def _sc_scatter(ef, dst_w):
  fn = pl.kernel(
      _sc_scatter_body,
      out_type=jax.ShapeDtypeStruct((NC, NGP, H), F32),
      mesh=_mesh(),
      scratch_types=[
          pltpu.VMEM_SHARED((NGP, H), F32),
          pltpu.VMEM((K, SCH, H), F32),
          pltpu.VMEM((K, SCH), jnp.int32),
          pltpu.SemaphoreType.DMA,
      ],
  )
  return fn(ef, dst_w)


def _node_body(g_ref, p0_ref, p1_ref, wn1a_ref, wn1b_ref, bn1_ref,
               wn2_ref, bn2_ref, gam_ref, bet_ref, out_ref):
  g = g_ref[...]
  agg = p0_ref[0] + p1_ref[0]
  h = (jnp.dot(g.astype(jnp.bfloat16), wn1a_ref[...],
               preferred_element_type=F32)
       + jnp.dot(agg.astype(jnp.bfloat16), wn1b_ref[...],
                 preferred_element_type=F32)
       + bn1_ref[...])
  h = h * jax.nn.sigmoid(h)
  h = jnp.dot(h.astype(jnp.bfloat16), wn2_ref[...],
              preferred_element_type=F32) + bn2_ref[...]
  mu = jnp.mean(h, axis=-1, keepdims=True)
  var = jnp.mean((h - mu) * (h - mu), axis=-1, keepdims=True)
  out_ref[...] = ((h - mu) * lax.rsqrt(var + 1e-5) * gam_ref[...]
                  + bet_ref[...] + g)


def _node_mlp(grid_nodes, partials, wn1a, wn1b, bn1, wn2, bn2, gam, bet):
  blk = 2000
  return pl.pallas_call(
      _node_body,
      grid=(NG // blk,),
      in_specs=[
          pl.BlockSpec((blk, H), lambda i: (i, 0)),
          pl.BlockSpec((1, blk, H), lambda i: (0, i, 0)),
          pl.BlockSpec((1, blk, H), lambda i: (1, i, 0)),
          pl.BlockSpec((H, H), lambda i: (0, 0)),
          pl.BlockSpec((H, H), lambda i: (0, 0)),
          pl.BlockSpec((1, H), lambda i: (0, 0)),
          pl.BlockSpec((H, H), lambda i: (0, 0)),
          pl.BlockSpec((1, H), lambda i: (0, 0)),
          pl.BlockSpec((1, H), lambda i: (0, 0)),
          pl.BlockSpec((1, H), lambda i: (0, 0)),
      ],
      out_specs=pl.BlockSpec((blk, H), lambda i: (i, 0)),
      out_shape=jax.ShapeDtypeStruct((NG, H), F32),
  )(grid_nodes, partials, partials, wn1a, wn1b, bn1, wn2, bn2, gam, bet)


def kernel(mesh2grid_edge_features, grid_node_features, mesh_node_features,
           mesh2grid_edge_indices_src, mesh2grid_edge_indices_dst,
           We1, be1, We2, be2, e_gamma, e_beta,
           Wn1, bn1, Wn2, bn2, n_gamma, n_beta):
  w1a = We1[:H]
  w1b = We1[H:2 * H]
  w1c = We1[2 * H:]
  wn1a = Wn1[:H]
  wn1b = Wn1[H:]
  r1 = lambda v: v.reshape(1, H)

  src_r = mesh2grid_edge_indices_src.reshape(NW, NCHUNK, CH)
  dst_r = mesh2grid_edge_indices_dst.reshape(NW, NCHUNK, CH)
  dst_w = mesh2grid_edge_indices_dst.reshape(NW, SNCHUNK // K, K, SCH)

  psrc, pdst = _project(mesh_node_features, grid_node_features,
                        w1b, w1c, r1(be1))
  gs, gd = _sc_gather(psrc, pdst, src_r, dst_r)
  bf16 = jnp.bfloat16
  ef = _edge_mlp(mesh2grid_edge_features, gs, gd, w1a.astype(bf16),
                 We2.astype(bf16), r1(be2), r1(e_gamma), r1(e_beta))
  partials = _sc_scatter(ef, dst_w)
  out = _node_mlp(grid_node_features, partials,
                  wn1a.astype(bf16), wn1b.astype(bf16), r1(bn1),
                  Wn2.astype(bf16), r1(bn2), r1(n_gamma), r1(n_beta))
  return out


# speedup vs baseline: 4.4140x; 1.1417x over previous
import functools

import jax
import jax.numpy as jnp
from jax import lax
from jax.experimental import pallas as pl
from jax.experimental.pallas import tpu as pltpu
from jax.experimental.pallas import tpu_sc as plsc

F32 = jnp.float32
BF16 = jnp.bfloat16
NG = 10000
NM = 10000
NE = 320000
NEH = NE // 2
H = 128

NC = 2
NS = 16
NW = NC * NS

EPW = NEH // NW
CH = 40
NCHUNK = EPW // CH
K = 5
SR = K * CH
NSUP = NCHUNK // K
NGP = 10240
ACC_PW = NGP // NS
EBLK = 1280


def _mesh():
  return plsc.VectorSubcoreMesh(core_axis_name="c", subcore_axis_name="s",
                                num_cores=NC, num_subcores=NS)


def _proj_body(mesh_ref, grid_ref, w1b_ref, w1c_ref, be1_ref,
               psrc_ref, pdst_ref):
  psrc_ref[...] = jnp.dot(mesh_ref[...], w1b_ref[...],
                          preferred_element_type=F32)
  pdst_ref[...] = jnp.dot(grid_ref[...], w1c_ref[...],
                          preferred_element_type=F32) + be1_ref[...]


def _project(mesh_nodes, grid_nodes, w1b, w1c, be1):
  blk = 2000
  return pl.pallas_call(
      _proj_body,
      grid=(NM // blk,),
      in_specs=[
          pl.BlockSpec((blk, H), lambda i: (i, 0)),
          pl.BlockSpec((blk, H), lambda i: (i, 0)),
          pl.BlockSpec((H, H), lambda i: (0, 0)),
          pl.BlockSpec((H, H), lambda i: (0, 0)),
          pl.BlockSpec((1, H), lambda i: (0, 0)),
      ],
      out_specs=[
          pl.BlockSpec((blk, H), lambda i: (i, 0)),
          pl.BlockSpec((blk, H), lambda i: (i, 0)),
      ],
      out_shape=[jax.ShapeDtypeStruct((NM, H), F32),
                 jax.ShapeDtypeStruct((NG, H), F32)],
  )(mesh_nodes, grid_nodes, w1b, w1c, be1)


def _sc_gather_body(psrc_hbm, pdst_hbm, srcr_hbm, dstr_hbm,
                    gs_hbm, gd_hbm,
                    idxs_v, idxd_v, rows_v, sem_g, sem_w):
  c = lax.axis_index("c")
  s = lax.axis_index("s")
  wid = c * NS + s
  ebase = wid * EPW
  pltpu.sync_copy(srcr_hbm.at[wid], idxs_v)
  pltpu.sync_copy(dstr_hbm.at[wid], idxd_v)

  def one_table(idx_v, table_hbm, out_hbm):
    def superchunk(m, carry):
      gets = []
      for k in range(K):
        cp = pltpu.make_async_copy(table_hbm.at[idx_v.at[m * K + k]],
                                   rows_v.at[pl.ds(k * CH, CH)], sem_g)
        cp.start()
        gets.append(cp)
      for cp in gets:
        cp.wait()
      pltpu.sync_copy(rows_v, out_hbm.at[pl.ds(ebase + m * SR, SR)])
      return carry
    lax.fori_loop(0, NSUP, superchunk, 0)

  one_table(idxs_v, psrc_hbm, gs_hbm)
  one_table(idxd_v, pdst_hbm, gd_hbm)


def _sc_gather(psrc, pdst, src_r, dst_r):
  fn = pl.kernel(
      _sc_gather_body,
      out_type=[jax.ShapeDtypeStruct((NEH, H), F32),
                jax.ShapeDtypeStruct((NEH, H), F32)],
      mesh=_mesh(),
      scratch_types=[
          pltpu.VMEM((NCHUNK, CH), jnp.int32),
          pltpu.VMEM((NCHUNK, CH), jnp.int32),
          pltpu.VMEM((SR, H), F32),
          pltpu.SemaphoreType.DMA,
          pltpu.SemaphoreType.DMA,
      ],
  )
  return fn(psrc, pdst, src_r, dst_r)


def _edge_body(e_ref, gs_ref, gd_ref, w1a_ref, w2_ref, be2_ref,
               gam_ref, bet_ref, out_ref):
  e = e_ref[...]
  h = jnp.dot(e.astype(BF16), w1a_ref[...], preferred_element_type=F32)
  h = h + gs_ref[...] + gd_ref[...]
  h = h * jax.nn.sigmoid(h)
  h = jnp.dot(h.astype(BF16), w2_ref[...],
              preferred_element_type=F32) + be2_ref[...]
  mu = jnp.mean(h, axis=-1, keepdims=True)
  var = jnp.mean((h - mu) * (h - mu), axis=-1, keepdims=True)
  out_ref[...] = ((h - mu) * lax.rsqrt(var + 1e-5) * gam_ref[...]
                  + bet_ref[...] + e)


def _edge_mlp(e_feats, half, gs, gd, w1a, w2, be2, gam, bet):
  base = half * (NEH // EBLK)
  return pl.pallas_call(
      _edge_body,
      grid=(NEH // EBLK,),
      in_specs=[
          pl.BlockSpec((EBLK, H), lambda i: (i + base, 0)),
          pl.BlockSpec((EBLK, H), lambda i: (i, 0)),
          pl.BlockSpec((EBLK, H), lambda i: (i, 0)),
          pl.BlockSpec((H, H), lambda i: (0, 0)),
          pl.BlockSpec((H, H), lambda i: (0, 0)),
          pl.BlockSpec((1, H), lambda i: (0, 0)),
          pl.BlockSpec((1, H), lambda i: (0, 0)),
          pl.BlockSpec((1, H), lambda i: (0, 0)),
      ],
      out_specs=pl.BlockSpec((EBLK, H), lambda i: (i, 0)),
      out_shape=jax.ShapeDtypeStruct((NEH, H), F32),
      compiler_params=pltpu.CompilerParams(
          dimension_semantics=("arbitrary",)),
  )(e_feats, gs, gd, w1a, w2, be2, gam, bet)


def _sc_scatter_body(ef_hbm, dstw_hbm, out_hbm,
                     acc_sh, val_v, idx_v, sem_in):
  c = lax.axis_index("c")
  s = lax.axis_index("s")
  wid = c * NS + s
  ebase = wid * EPW

  zero = jnp.zeros((16,), F32)

  def zrow(r, carry):
    for k in range(H // 16):
      val_v[r, pl.ds(16 * k, 16)] = zero
    return carry
  lax.fori_loop(0, CH, zrow, 0)
  rbase = s * ACC_PW
  for z in range(ACC_PW // CH):
    pltpu.sync_copy(val_v.at[pl.ds(0, CH)],
                    acc_sh.at[pl.ds(rbase + z * CH, CH)])
  plsc.subcore_barrier()

  def superchunk(m, carry):
    icp = pltpu.make_async_copy(dstw_hbm.at[wid, m], idx_v, sem_in)
    icp.start()
    vcp = pltpu.make_async_copy(ef_hbm.at[pl.ds(ebase + m * SR, SR)],
                                val_v, sem_in)
    vcp.start()
    icp.wait()
    vcp.wait()
    for k in range(K):
      pltpu.sync_copy(val_v.at[pl.ds(k * CH, CH)],
                      acc_sh.at[idx_v.at[k]], add=True)
    return carry
  lax.fori_loop(0, NSUP, superchunk, 0)
  plsc.subcore_barrier()

  for z in range(ACC_PW // CH):
    pltpu.sync_copy(acc_sh.at[pl.ds(rbase + z * CH, CH)],
                    val_v.at[pl.ds(0, CH)])
    pltpu.sync_copy(val_v.at[pl.ds(0, CH)],
                    out_hbm.at[c, pl.ds(rbase + z * CH, CH)])


def _sc_scatter(ef, dst_w):
  fn = pl.kernel(
      _sc_scatter_body,
      out_type=jax.ShapeDtypeStruct((NC, NGP, H), F32),
      mesh=_mesh(),
      scratch_types=[
          pltpu.VMEM_SHARED((NGP, H), F32),
          pltpu.VMEM((SR, H), F32),
          pltpu.VMEM((K, CH), jnp.int32),
          pltpu.SemaphoreType.DMA,
      ],
  )
  return fn(ef, dst_w)


def _node_body(g_ref, p0_ref, p1_ref, p2_ref, p3_ref,
               wn1a_ref, wn1b_ref, bn1_ref,
               wn2_ref, bn2_ref, gam_ref, bet_ref, out_ref):
  g = g_ref[...]
  agg = (p0_ref[0] + p1_ref[0]) + (p2_ref[0] + p3_ref[0])
  h = (jnp.dot(g.astype(BF16), wn1a_ref[...], preferred_element_type=F32)
       + jnp.dot(agg.astype(BF16), wn1b_ref[...],
                 preferred_element_type=F32)
       + bn1_ref[...])
  h = h * jax.nn.sigmoid(h)
  h = jnp.dot(h.astype(BF16), wn2_ref[...],
              preferred_element_type=F32) + bn2_ref[...]
  mu = jnp.mean(h, axis=-1, keepdims=True)
  var = jnp.mean((h - mu) * (h - mu), axis=-1, keepdims=True)
  out_ref[...] = ((h - mu) * lax.rsqrt(var + 1e-5) * gam_ref[...]
                  + bet_ref[...] + g)


def _node_mlp(grid_nodes, pa, pb, wn1a, wn1b, bn1, wn2, bn2, gam, bet):
  blk = 2000
  return pl.pallas_call(
      _node_body,
      grid=(NG // blk,),
      in_specs=[
          pl.BlockSpec((blk, H), lambda i: (i, 0)),
          pl.BlockSpec((1, blk, H), lambda i: (0, i, 0)),
          pl.BlockSpec((1, blk, H), lambda i: (1, i, 0)),
          pl.BlockSpec((1, blk, H), lambda i: (0, i, 0)),
          pl.BlockSpec((1, blk, H), lambda i: (1, i, 0)),
          pl.BlockSpec((H, H), lambda i: (0, 0)),
          pl.BlockSpec((H, H), lambda i: (0, 0)),
          pl.BlockSpec((1, H), lambda i: (0, 0)),
          pl.BlockSpec((H, H), lambda i: (0, 0)),
          pl.BlockSpec((1, H), lambda i: (0, 0)),
          pl.BlockSpec((1, H), lambda i: (0, 0)),
          pl.BlockSpec((1, H), lambda i: (0, 0)),
      ],
      out_specs=pl.BlockSpec((blk, H), lambda i: (i, 0)),
      out_shape=jax.ShapeDtypeStruct((NG, H), F32),
  )(grid_nodes, pa, pa, pb, pb, wn1a, wn1b, bn1, wn2, bn2, gam, bet)


def kernel(mesh2grid_edge_features, grid_node_features, mesh_node_features,
           mesh2grid_edge_indices_src, mesh2grid_edge_indices_dst,
           We1, be1, We2, be2, e_gamma, e_beta,
           Wn1, bn1, Wn2, bn2, n_gamma, n_beta):
  w1a = We1[:H]
  w1b = We1[H:2 * H]
  w1c = We1[2 * H:]
  wn1a = Wn1[:H]
  wn1b = Wn1[H:]
  r1 = lambda v: v.reshape(1, H)

  src = mesh2grid_edge_indices_src
  dst = mesh2grid_edge_indices_dst
  src_r = [src[:NEH].reshape(NW, NCHUNK, CH),
           src[NEH:].reshape(NW, NCHUNK, CH)]
  dst_r = [dst[:NEH].reshape(NW, NCHUNK, CH),
           dst[NEH:].reshape(NW, NCHUNK, CH)]
  dst_w = [dst[:NEH].reshape(NW, NSUP, K, CH),
           dst[NEH:].reshape(NW, NSUP, K, CH)]

  psrc, pdst = _project(mesh_node_features, grid_node_features,
                        w1b, w1c, r1(be1))

  w1a_b = w1a.astype(BF16)
  w2_b = We2.astype(BF16)
  e_args = (w1a_b, w2_b, r1(be2), r1(e_gamma), r1(e_beta))

  g0s, g0d = _sc_gather(psrc, pdst, src_r[0], dst_r[0])
  ef0 = _edge_mlp(mesh2grid_edge_features, 0, g0s, g0d, *e_args)
  g1s, g1d = _sc_gather(psrc, pdst, src_r[1], dst_r[1])
  ef1 = _edge_mlp(mesh2grid_edge_features, 1, g1s, g1d, *e_args)
  pa = _sc_scatter(ef0, dst_w[0])
  pb = _sc_scatter(ef1, dst_w[1])

  out = _node_mlp(grid_node_features, pa, pb,
                  wn1a.astype(BF16), wn1b.astype(BF16), r1(bn1),
                  Wn2.astype(BF16), r1(bn2), r1(n_gamma), r1(n_beta))
  return out


# speedup vs baseline: 5.0892x; 1.1529x over previous
import functools

import jax
import jax.numpy as jnp
from jax import lax
from jax.experimental import pallas as pl
from jax.experimental.pallas import tpu as pltpu
from jax.experimental.pallas import tpu_sc as plsc

F32 = jnp.float32
BF16 = jnp.bfloat16
NG = 10000
NM = 10000
NE = 320000
NEH = NE // 2
H = 128

NC = 2
NS = 16
NW = NC * NS

EPW = NEH // NW
CH = 40
NCHUNK = EPW // CH
K = 5
SR = K * CH
NSUP = NCHUNK // K
NGP = 10240
ACC_PW = NGP // NS
EBLK = 1280


def _mesh():
  return plsc.VectorSubcoreMesh(core_axis_name="c", subcore_axis_name="s",
                                num_cores=NC, num_subcores=NS)


def _proj_body(mesh_ref, grid_ref, w1b_ref, w1c_ref, be1_ref,
               psrc_ref, pdst_ref):
  psrc_ref[...] = jnp.dot(mesh_ref[...], w1b_ref[...],
                          preferred_element_type=F32)
  pdst_ref[...] = jnp.dot(grid_ref[...], w1c_ref[...],
                          preferred_element_type=F32) + be1_ref[...]


def _project(mesh_nodes, grid_nodes, w1b, w1c, be1):
  blk = 2000
  return pl.pallas_call(
      _proj_body,
      grid=(NM // blk,),
      in_specs=[
          pl.BlockSpec((blk, H), lambda i: (i, 0)),
          pl.BlockSpec((blk, H), lambda i: (i, 0)),
          pl.BlockSpec((H, H), lambda i: (0, 0)),
          pl.BlockSpec((H, H), lambda i: (0, 0)),
          pl.BlockSpec((1, H), lambda i: (0, 0)),
      ],
      out_specs=[
          pl.BlockSpec((blk, H), lambda i: (i, 0)),
          pl.BlockSpec((blk, H), lambda i: (i, 0)),
      ],
      out_shape=[jax.ShapeDtypeStruct((NM, H), F32),
                 jax.ShapeDtypeStruct((NG, H), F32)],
  )(mesh_nodes, grid_nodes, w1b, w1c, be1)


def _sc_gather_body(psrc_hbm, pdst_hbm, srcr_hbm, dstr_hbm,
                    gs_hbm,
                    idxs_v, idxd_v, rows_v, sem_g, sem_w):
  c = lax.axis_index("c")
  s = lax.axis_index("s")
  wid = c * NS + s
  ebase = wid * EPW
  pltpu.sync_copy(srcr_hbm.at[wid], idxs_v)
  pltpu.sync_copy(dstr_hbm.at[wid], idxd_v)

  def superchunk(m, carry):
    gets = []
    for k in range(K):
      cp = pltpu.make_async_copy(psrc_hbm.at[idxs_v.at[m * K + k]],
                                 rows_v.at[pl.ds(k * CH, CH)], sem_g)
      cp.start()
      gets.append(cp)
    for cp in gets:
      cp.wait()
    adds = []
    for k in range(K):
      cp = pltpu.async_copy(pdst_hbm.at[idxd_v.at[m * K + k]],
                            rows_v.at[pl.ds(k * CH, CH)], sem_g,
                            add=True)
      adds.append(cp)
    for cp in adds:
      cp.wait()
    pltpu.sync_copy(rows_v, gs_hbm.at[pl.ds(ebase + m * SR, SR)])
    return carry
  lax.fori_loop(0, NSUP, superchunk, 0)


def _sc_gather(psrc, pdst, src_r, dst_r):
  fn = pl.kernel(
      _sc_gather_body,
      out_type=jax.ShapeDtypeStruct((NEH, H), F32),
      mesh=_mesh(),
      scratch_types=[
          pltpu.VMEM((NCHUNK, CH), jnp.int32),
          pltpu.VMEM((NCHUNK, CH), jnp.int32),
          pltpu.VMEM((SR, H), F32),
          pltpu.SemaphoreType.DMA,
          pltpu.SemaphoreType.DMA,
      ],
  )
  return fn(psrc, pdst, src_r, dst_r)


def _edge_body(e_ref, gs_ref, w1a_ref, w2_ref, be2_ref,
               gam_ref, bet_ref, out_ref):
  e = e_ref[...]
  h = jnp.dot(e.astype(BF16), w1a_ref[...], preferred_element_type=F32)
  h = h + gs_ref[...]
  h = h * jax.nn.sigmoid(h)
  h = jnp.dot(h.astype(BF16), w2_ref[...],
              preferred_element_type=F32) + be2_ref[...]
  mu = jnp.mean(h, axis=-1, keepdims=True)
  var = jnp.mean((h - mu) * (h - mu), axis=-1, keepdims=True)
  out_ref[...] = ((h - mu) * lax.rsqrt(var + 1e-5) * gam_ref[...]
                  + bet_ref[...] + e)


def _edge_mlp(e_feats, half, gs, w1a, w2, be2, gam, bet):
  base = half * (NEH // EBLK)
  return pl.pallas_call(
      _edge_body,
      grid=(NEH // EBLK,),
      in_specs=[
          pl.BlockSpec((EBLK, H), lambda i: (i + base, 0)),
          pl.BlockSpec((EBLK, H), lambda i: (i, 0)),
          pl.BlockSpec((H, H), lambda i: (0, 0)),
          pl.BlockSpec((H, H), lambda i: (0, 0)),
          pl.BlockSpec((1, H), lambda i: (0, 0)),
          pl.BlockSpec((1, H), lambda i: (0, 0)),
          pl.BlockSpec((1, H), lambda i: (0, 0)),
      ],
      out_specs=pl.BlockSpec((EBLK, H), lambda i: (i, 0)),
      out_shape=jax.ShapeDtypeStruct((NEH, H), F32),
      compiler_params=pltpu.CompilerParams(
          dimension_semantics=("arbitrary",)),
  )(e_feats, gs, w1a, w2, be2, gam, bet)


def _sc_scatter_body(ef_hbm, dstw_hbm, out_hbm,
                     acc_sh, val_v, idx_v, sem_in):
  c = lax.axis_index("c")
  s = lax.axis_index("s")
  wid = c * NS + s
  ebase = wid * EPW

  zero = jnp.zeros((16,), F32)

  def zrow(r, carry):
    for k in range(H // 16):
      val_v[r, pl.ds(16 * k, 16)] = zero
    return carry
  lax.fori_loop(0, CH, zrow, 0)
  rbase = s * ACC_PW
  for z in range(ACC_PW // CH):
    pltpu.sync_copy(val_v.at[pl.ds(0, CH)],
                    acc_sh.at[pl.ds(rbase + z * CH, CH)])
  plsc.subcore_barrier()

  def superchunk(m, carry):
    icp = pltpu.make_async_copy(dstw_hbm.at[wid, m], idx_v, sem_in)
    icp.start()
    vcp = pltpu.make_async_copy(ef_hbm.at[pl.ds(ebase + m * SR, SR)],
                                val_v, sem_in)
    vcp.start()
    icp.wait()
    vcp.wait()
    for k in range(K):
      pltpu.sync_copy(val_v.at[pl.ds(k * CH, CH)],
                      acc_sh.at[idx_v.at[k]], add=True)
    return carry
  lax.fori_loop(0, NSUP, superchunk, 0)
  plsc.subcore_barrier()

  for z in range(ACC_PW // CH):
    pltpu.sync_copy(acc_sh.at[pl.ds(rbase + z * CH, CH)],
                    val_v.at[pl.ds(0, CH)])
    pltpu.sync_copy(val_v.at[pl.ds(0, CH)],
                    out_hbm.at[c, pl.ds(rbase + z * CH, CH)])


def _sc_scatter(ef, dst_w):
  fn = pl.kernel(
      _sc_scatter_body,
      out_type=jax.ShapeDtypeStruct((NC, NGP, H), F32),
      mesh=_mesh(),
      scratch_types=[
          pltpu.VMEM_SHARED((NGP, H), F32),
          pltpu.VMEM((SR, H), F32),
          pltpu.VMEM((K, CH), jnp.int32),
          pltpu.SemaphoreType.DMA,
      ],
  )
  return fn(ef, dst_w)


def _node_body(g_ref, p0_ref, p1_ref, p2_ref, p3_ref,
               wn1a_ref, wn1b_ref, bn1_ref,
               wn2_ref, bn2_ref, gam_ref, bet_ref, out_ref):
  g = g_ref[...]
  agg = (p0_ref[0] + p1_ref[0]) + (p2_ref[0] + p3_ref[0])
  h = (jnp.dot(g.astype(BF16), wn1a_ref[...], preferred_element_type=F32)
       + jnp.dot(agg.astype(BF16), wn1b_ref[...],
                 preferred_element_type=F32)
       + bn1_ref[...])
  h = h * jax.nn.sigmoid(h)
  h = jnp.dot(h.astype(BF16), wn2_ref[...],
              preferred_element_type=F32) + bn2_ref[...]
  mu = jnp.mean(h, axis=-1, keepdims=True)
  var = jnp.mean((h - mu) * (h - mu), axis=-1, keepdims=True)
  out_ref[...] = ((h - mu) * lax.rsqrt(var + 1e-5) * gam_ref[...]
                  + bet_ref[...] + g)


def _node_mlp(grid_nodes, pa, pb, wn1a, wn1b, bn1, wn2, bn2, gam, bet):
  blk = 2000
  return pl.pallas_call(
      _node_body,
      grid=(NG // blk,),
      in_specs=[
          pl.BlockSpec((blk, H), lambda i: (i, 0)),
          pl.BlockSpec((1, blk, H), lambda i: (0, i, 0)),
          pl.BlockSpec((1, blk, H), lambda i: (1, i, 0)),
          pl.BlockSpec((1, blk, H), lambda i: (0, i, 0)),
          pl.BlockSpec((1, blk, H), lambda i: (1, i, 0)),
          pl.BlockSpec((H, H), lambda i: (0, 0)),
          pl.BlockSpec((H, H), lambda i: (0, 0)),
          pl.BlockSpec((1, H), lambda i: (0, 0)),
          pl.BlockSpec((H, H), lambda i: (0, 0)),
          pl.BlockSpec((1, H), lambda i: (0, 0)),
          pl.BlockSpec((1, H), lambda i: (0, 0)),
          pl.BlockSpec((1, H), lambda i: (0, 0)),
      ],
      out_specs=pl.BlockSpec((blk, H), lambda i: (i, 0)),
      out_shape=jax.ShapeDtypeStruct((NG, H), F32),
  )(grid_nodes, pa, pa, pb, pb, wn1a, wn1b, bn1, wn2, bn2, gam, bet)


def kernel(mesh2grid_edge_features, grid_node_features, mesh_node_features,
           mesh2grid_edge_indices_src, mesh2grid_edge_indices_dst,
           We1, be1, We2, be2, e_gamma, e_beta,
           Wn1, bn1, Wn2, bn2, n_gamma, n_beta):
  w1a = We1[:H]
  w1b = We1[H:2 * H]
  w1c = We1[2 * H:]
  wn1a = Wn1[:H]
  wn1b = Wn1[H:]
  r1 = lambda v: v.reshape(1, H)

  src = mesh2grid_edge_indices_src
  dst = mesh2grid_edge_indices_dst
  src_r = [src[:NEH].reshape(NW, NCHUNK, CH),
           src[NEH:].reshape(NW, NCHUNK, CH)]
  dst_r = [dst[:NEH].reshape(NW, NCHUNK, CH),
           dst[NEH:].reshape(NW, NCHUNK, CH)]
  dst_w = [dst[:NEH].reshape(NW, NSUP, K, CH),
           dst[NEH:].reshape(NW, NSUP, K, CH)]

  psrc, pdst = _project(mesh_node_features, grid_node_features,
                        w1b, w1c, r1(be1))

  w1a_b = w1a.astype(BF16)
  w2_b = We2.astype(BF16)
  e_args = (w1a_b, w2_b, r1(be2), r1(e_gamma), r1(e_beta))

  g0 = _sc_gather(psrc, pdst, src_r[0], dst_r[0])
  ef0 = _edge_mlp(mesh2grid_edge_features, 0, g0, *e_args)
  g1 = _sc_gather(psrc, pdst, src_r[1], dst_r[1])
  ef1 = _edge_mlp(mesh2grid_edge_features, 1, g1, *e_args)
  pa = _sc_scatter(ef0, dst_w[0])
  pb = _sc_scatter(ef1, dst_w[1])

  out = _node_mlp(grid_node_features, pa, pb,
                  wn1a.astype(BF16), wn1b.astype(BF16), r1(bn1),
                  Wn2.astype(BF16), r1(bn2), r1(n_gamma), r1(n_beta))
  return out
